# baseline (speedup 1.0000x reference)
import functools

import jax
import jax.numpy as jnp
from jax.experimental import pallas as pl
from jax.experimental.pallas import tpu as pltpu


def _rup(x, m):
    return ((x + m - 1) // m) * m


def _cols9_dot(xs_ref, cols_ref, w_ref, mask_ref, *, C, HWo, Wo, HP, Mp, full):
    G = HWo
    Wn = Mp + 2 * G
    base = HP - G
    for kh in range(3):
        for kw in range(3):
            t = 3 * kh + kw
            sh = (kh - 1) * Wo + (kw - 1)
            tap = xs_ref[:, base + sh:base + sh + Wn]
            if not (full and t == 4):
                tap = tap * mask_ref[t:t + 1, :]
            cols_ref[t * C:(t + 1) * C, :] = tap
    acc = jnp.dot(w_ref[0], cols_ref[:, 0:Mp],
                  preferred_element_type=jnp.float32)
    acc += jnp.dot(w_ref[1], cols_ref[:, G:G + Mp],
                   preferred_element_type=jnp.float32)
    acc += jnp.dot(w_ref[2], cols_ref[:, 2 * G:2 * G + Mp],
                   preferred_element_type=jnp.float32)
    return acc


def _pool_conv1_kernel(x_ref, sel_ref, w_ref, mask_ref,
                       y_ref, ssum_ref, ssq_ref,
                       pool_ref, xs_ref, cols_ref, *,
                       C, Do, HW_in, W_in, HWo, Wo, HP, M, Mp, full):
    for do in range(Do):
        a = jnp.maximum(x_ref[0, :, 2 * do, :],
                        x_ref[0, :, 2 * do + 1, :])
        b = jnp.maximum(a, jnp.concatenate([a[:, W_in:], a[:, :W_in]], axis=1))
        c = jnp.maximum(b, jnp.concatenate([b[:, 1:], b[:, :1]], axis=1))
        pool_ref[do * C:(do + 1) * C, :] = c.astype(jnp.bfloat16)
    p = jnp.dot(pool_ref[...], sel_ref[...],
                preferred_element_type=jnp.float32)
    p = p.astype(jnp.bfloat16)

    L = xs_ref.shape[1]
    xs_ref[:, :HP] = jnp.zeros((C, HP), jnp.bfloat16)
    for do in range(Do):
        xs_ref[:, HP + do * HWo:HP + (do + 1) * HWo] = p[do * C:(do + 1) * C, :]
    xs_ref[:, HP + M:] = jnp.zeros((C, L - HP - M), jnp.bfloat16)

    acc = _cols9_dot(xs_ref, cols_ref, w_ref, mask_ref,
                     C=C, HWo=HWo, Wo=Wo, HP=HP, Mp=Mp, full=full)
    cout = y_ref.shape[1]
    y_ref[0, :, :HP] = jnp.zeros((cout, HP), jnp.bfloat16)
    y_ref[0, :, HP:HP + Mp] = acc.astype(jnp.bfloat16)
    y_ref[0, :, HP + Mp:] = jnp.zeros((cout, L - HP - Mp), jnp.bfloat16)
    ssum_ref[0] = jnp.sum(acc, axis=1, keepdims=True)
    ssq_ref[0] = jnp.sum(acc * acc, axis=1, keepdims=True)


def _conv2_kernel(y1_ref, scale_ref, shift_ref, valid_ref, w_ref, mask_ref,
                  y2_ref, ssum_ref, ssq_ref,
                  xs_ref, cols_ref, *, C, HWo, Wo, HP, Mp, full):
    yv = y1_ref[0].astype(jnp.float32)
    act = jnp.maximum(yv * scale_ref[...] + shift_ref[...], 0.0)
    xs_ref[...] = (act * valid_ref[...]).astype(jnp.bfloat16)
    acc = _cols9_dot(xs_ref, cols_ref, w_ref, mask_ref,
                     C=C, HWo=HWo, Wo=Wo, HP=HP, Mp=Mp, full=full)
    y2_ref[0] = acc.astype(jnp.bfloat16)
    ssum_ref[0] = jnp.sum(acc, axis=1, keepdims=True)
    ssq_ref[0] = jnp.sum(acc * acc, axis=1, keepdims=True)


def _bn_relu_out_kernel(y_ref, scale_ref, shift_ref, o_ref):
    o_ref[0] = jnp.maximum(
        y_ref[0].astype(jnp.float32) * scale_ref[...] + shift_ref[...], 0.0)


def _fold_w9(w):
    cout, cin = w.shape[0], w.shape[1]
    wt = jnp.transpose(w.astype(jnp.float32), (2, 3, 4, 0, 1))
    wt = jnp.transpose(wt.reshape(3, 9, cout, cin), (0, 2, 1, 3))
    return wt.reshape(3, cout, 9 * cin).astype(jnp.bfloat16)


def _fold_bn(ssum, ssq, count, gamma, beta, eps=1e-5):
    s = jnp.sum(ssum[:, :, 0], axis=0)
    sq = jnp.sum(ssq[:, :, 0], axis=0)
    mean = s / count
    var = sq / count - mean * mean
    inv = gamma / jnp.sqrt(var + eps)
    scale = inv.reshape(-1, 1).astype(jnp.float32)
    shift = (beta - mean * inv).reshape(-1, 1).astype(jnp.float32)
    return scale, shift


def kernel(x, w1, g1, be1, w2, g2, be2):
    N, Cin, D, H, W = x.shape
    C1, C2 = w1.shape[0], w2.shape[0]
    assert Cin % 8 == 0 and C1 % 8 == 0 and C2 % 8 == 0 and N % 2 == 0
    assert D % 2 == 0 and H % 2 == 0 and W % 2 == 0
    Do, Ho, Wo = D // 2, H // 2, W // 2
    HWo = Ho * Wo
    M = Do * HWo
    Mp = _rup(M, 128)
    HP = _rup(HWo + Wo + 1, 128)
    L = HP + Mp + HP
    HW_in = H * W
    Wn = Mp + 2 * HWo
    full = (M == Mp)
    assert full
    assert HP >= HWo + Wo + 1 and L - HP - Mp >= HWo + Wo + 1

    xr = x.reshape(N, Cin, D, HW_in)

    l_idx = jnp.arange(HW_in)[:, None]
    k_idx = jnp.arange(HWo)[None, :]
    sel = (l_idx == 2 * W * (k_idx // Wo) + 2 * (k_idx % Wo)).astype(jnp.bfloat16)
    j = jnp.arange(Wn)
    r = (j - HWo) % HWo
    w_i = r % Wo
    h_i = r // Wo
    rows = []
    for kh in range(3):
        for kw in range(3):
            ok = ((h_i + kh - 1 >= 0) & (h_i + kh - 1 < Ho)
                  & (w_i + kw - 1 >= 0) & (w_i + kw - 1 < Wo))
            rows.append(ok)
    mask = jnp.stack(rows, axis=0).astype(jnp.bfloat16)
    lane = jnp.arange(L)
    valid = ((lane >= HP) & (lane < HP + M)).astype(jnp.float32).reshape(1, L)

    w1f = _fold_w9(w1)
    w2f = _fold_w9(w2)

    k1 = functools.partial(_pool_conv1_kernel, C=Cin, Do=Do, HW_in=HW_in,
                           W_in=W, HWo=HWo, Wo=Wo, HP=HP, M=M, Mp=Mp, full=full)
    y1, s1, q1 = pl.pallas_call(
        k1,
        out_shape=(jax.ShapeDtypeStruct((N, C1, L), jnp.bfloat16),
                   jax.ShapeDtypeStruct((N, C1, 1), jnp.float32),
                   jax.ShapeDtypeStruct((N, C1, 1), jnp.float32)),
        grid=(N,),
        in_specs=[pl.BlockSpec((1, Cin, D, HW_in), lambda n: (n, 0, 0, 0)),
                  pl.BlockSpec((HW_in, HWo), lambda n: (0, 0)),
                  pl.BlockSpec((3, C1, 9 * Cin), lambda n: (0, 0, 0)),
                  pl.BlockSpec((9, Wn), lambda n: (0, 0))],
        out_specs=(pl.BlockSpec((1, C1, L), lambda n: (n, 0, 0)),
                   pl.BlockSpec((1, C1, 1), lambda n: (n, 0, 0)),
                   pl.BlockSpec((1, C1, 1), lambda n: (n, 0, 0))),
        scratch_shapes=[pltpu.VMEM((Do * Cin, HW_in), jnp.bfloat16),
                        pltpu.VMEM((Cin, L), jnp.bfloat16),
                        pltpu.VMEM((9 * Cin, Wn), jnp.bfloat16)],
        compiler_params=pltpu.CompilerParams(
            dimension_semantics=("arbitrary",)),
    )(xr, sel, w1f, mask)
    sc1, sh1 = _fold_bn(s1, q1, N * M, g1, be1)

    k2 = functools.partial(_conv2_kernel, C=C1, HWo=HWo, Wo=Wo, HP=HP, Mp=Mp,
                           full=full)
    y2, s2, q2 = pl.pallas_call(
        k2,
        out_shape=(jax.ShapeDtypeStruct((N, C2, Mp), jnp.bfloat16),
                   jax.ShapeDtypeStruct((N, C2, 1), jnp.float32),
                   jax.ShapeDtypeStruct((N, C2, 1), jnp.float32)),
        grid=(N,),
        in_specs=[pl.BlockSpec((1, C1, L), lambda n: (n, 0, 0)),
                  pl.BlockSpec((C1, 1), lambda n: (0, 0)),
                  pl.BlockSpec((C1, 1), lambda n: (0, 0)),
                  pl.BlockSpec((1, L), lambda n: (0, 0)),
                  pl.BlockSpec((3, C2, 9 * C1), lambda n: (0, 0, 0)),
                  pl.BlockSpec((9, Wn), lambda n: (0, 0))],
        out_specs=(pl.BlockSpec((1, C2, Mp), lambda n: (n, 0, 0)),
                   pl.BlockSpec((1, C2, 1), lambda n: (n, 0, 0)),
                   pl.BlockSpec((1, C2, 1), lambda n: (n, 0, 0))),
        scratch_shapes=[pltpu.VMEM((C1, L), jnp.bfloat16),
                        pltpu.VMEM((9 * C1, Wn), jnp.bfloat16)],
        compiler_params=pltpu.CompilerParams(
            dimension_semantics=("arbitrary",)),
    )(y1, sc1, sh1, valid, w2f, mask)
    sc2, sh2 = _fold_bn(s2, q2, N * M, g2, be2)

    out = pl.pallas_call(
        _bn_relu_out_kernel,
        out_shape=jax.ShapeDtypeStruct((N, C2, Mp), jnp.float32),
        grid=(N,),
        in_specs=[pl.BlockSpec((1, C2, Mp), lambda n: (n, 0, 0)),
                  pl.BlockSpec((C2, 1), lambda n: (0, 0)),
                  pl.BlockSpec((C2, 1), lambda n: (0, 0))],
        out_specs=pl.BlockSpec((1, C2, Mp), lambda n: (n, 0, 0)),
        compiler_params=pltpu.CompilerParams(
            dimension_semantics=("arbitrary",)),
    )(y2, sc2, sh2)
    return out[:, :, :M].reshape(N, C2, Do, Ho, Wo)


# speedup vs baseline: 1.2914x; 1.2914x over previous
import functools

import jax
import jax.numpy as jnp
from jax.experimental import pallas as pl
from jax.experimental.pallas import tpu as pltpu


def _rup(x, m):
    return ((x + m - 1) // m) * m


def _cols9_dot(xs_ref, cols_ref, w_ref, mask_ref, *, C, HWo, Wo, HP, Mp):
    G = HWo
    Wn = Mp + 2 * G
    base = HP - G
    for kh in range(3):
        for kw in range(3):
            t = 3 * kh + kw
            sh = (kh - 1) * Wo + (kw - 1)
            tap = xs_ref[:, base + sh:base + sh + Wn]
            if t != 4:
                tap = tap * mask_ref[t:t + 1, :]
            cols_ref[t * C:(t + 1) * C, :] = tap
    acc = jnp.dot(w_ref[0], cols_ref[:, 0:Mp],
                  preferred_element_type=jnp.float32)
    acc += jnp.dot(w_ref[1], cols_ref[:, G:G + Mp],
                   preferred_element_type=jnp.float32)
    acc += jnp.dot(w_ref[2], cols_ref[:, 2 * G:2 * G + Mp],
                   preferred_element_type=jnp.float32)
    return acc


def _fused_convs_kernel(x_ref, sel_ref, w1_ref, w2_ref, mask_ref, valid_ref,
                        g1_ref, b1_ref,
                        y2_ref, s2_ref, q2_ref,
                        pool_ref, xs1_ref, xs2_ref, cols1_ref, cols2_ref,
                        y1s_ref, st1_ref, fold_ref, *,
                        Cin, C1, Do, HW_in, W_in, HWo, Wo, HP, M, Mp, NB, eps):
    s = pl.program_id(0)
    i = pl.program_id(1)
    L = xs1_ref.shape[1]

    @pl.when(s == 0)
    def _conv1_stage():
        for do in range(Do):
            v = x_ref[0, :, 2 * HW_in * do:2 * HW_in * (do + 1)]
            a = jnp.maximum(v[:, :HW_in], v[:, HW_in:])
            b = jnp.maximum(a, jnp.concatenate([a[:, W_in:], a[:, :W_in]], 1))
            c = jnp.maximum(b, jnp.concatenate([b[:, 1:], b[:, :1]], 1))
            pool_ref[do * Cin:(do + 1) * Cin, :] = c.astype(jnp.bfloat16)
        p = jnp.dot(pool_ref[...], sel_ref[...],
                    preferred_element_type=jnp.float32)
        p = p.astype(jnp.bfloat16)

        xs1_ref[:, :HP] = jnp.zeros((Cin, HP), jnp.bfloat16)
        for do in range(Do):
            xs1_ref[:, HP + do * HWo:HP + (do + 1) * HWo] = \
                p[do * Cin:(do + 1) * Cin, :]
        xs1_ref[:, HP + M:] = jnp.zeros((Cin, L - HP - M), jnp.bfloat16)

        acc = _cols9_dot(xs1_ref, cols1_ref, w1_ref, mask_ref,
                         C=Cin, HWo=HWo, Wo=Wo, HP=HP, Mp=Mp)
        row = pl.ds(pl.multiple_of(i * C1, C1), C1)
        y1s_ref[row, :HP] = jnp.zeros((C1, HP), jnp.bfloat16)
        y1s_ref[row, HP:HP + Mp] = acc.astype(jnp.bfloat16)
        y1s_ref[row, HP + Mp:] = jnp.zeros((C1, L - HP - Mp), jnp.bfloat16)

        @pl.when(i == 0)
        def _zero_stats():
            st1_ref[...] = jnp.zeros_like(st1_ref)
        st1_ref[:, 0:1] += jnp.sum(acc, axis=1, keepdims=True)
        st1_ref[:, 1:2] += jnp.sum(acc * acc, axis=1, keepdims=True)

    @pl.when(s == 1)
    def _conv2_stage():
        @pl.when(i == 0)
        def _fold_bn1():
            cnt = float(NB * M)
            mean = st1_ref[:, 0:1] / cnt
            var = st1_ref[:, 1:2] / cnt - mean * mean
            inv = g1_ref[...] / jnp.sqrt(var + eps)
            fold_ref[:, 0:1] = inv
            fold_ref[:, 1:2] = b1_ref[...] - mean * inv

        row = pl.ds(pl.multiple_of(i * C1, C1), C1)
        yv = y1s_ref[row, :].astype(jnp.float32)
        act = jnp.maximum(yv * fold_ref[:, 0:1] + fold_ref[:, 1:2], 0.0)
        xs2_ref[...] = (act * valid_ref[...]).astype(jnp.bfloat16)
        acc = _cols9_dot(xs2_ref, cols2_ref, w2_ref, mask_ref,
                         C=C1, HWo=HWo, Wo=Wo, HP=HP, Mp=Mp)
        y2_ref[0] = acc.astype(jnp.bfloat16)
        s2_ref[0] = jnp.sum(acc, axis=1, keepdims=True)
        q2_ref[0] = jnp.sum(acc * acc, axis=1, keepdims=True)


def _bn_relu_out_kernel(y_ref, scale_ref, shift_ref, o_ref):
    o_ref[0] = jnp.maximum(
        y_ref[0].astype(jnp.float32) * scale_ref[...] + shift_ref[...], 0.0)


def _fold_w9(w):
    cout, cin = w.shape[0], w.shape[1]
    wt = jnp.transpose(w.astype(jnp.float32), (2, 3, 4, 0, 1))
    wt = jnp.transpose(wt.reshape(3, 9, cout, cin), (0, 2, 1, 3))
    return wt.reshape(3, cout, 9 * cin).astype(jnp.bfloat16)


def _fold_bn(ssum, ssq, count, gamma, beta, eps=1e-5):
    s = jnp.sum(ssum[:, :, 0], axis=0)
    sq = jnp.sum(ssq[:, :, 0], axis=0)
    mean = s / count
    var = sq / count - mean * mean
    inv = gamma / jnp.sqrt(var + eps)
    scale = inv.reshape(-1, 1).astype(jnp.float32)
    shift = (beta - mean * inv).reshape(-1, 1).astype(jnp.float32)
    return scale, shift


def kernel(x, w1, g1, be1, w2, g2, be2):
    N, Cin, D, H, W = x.shape
    C1, C2 = w1.shape[0], w2.shape[0]
    assert Cin % 8 == 0 and C1 % 8 == 0 and C2 % 8 == 0
    assert D % 2 == 0 and H % 2 == 0 and W % 2 == 0
    Do, Ho, Wo = D // 2, H // 2, W // 2
    HWo = Ho * Wo
    M = Do * HWo
    Mp = _rup(M, 128)
    HP = _rup(HWo + Wo + 1, 128)
    L = HP + Mp + HP
    HW_in = H * W
    Wn = Mp + 2 * HWo
    assert M == Mp
    assert HP >= HWo + Wo + 1 and L - HP - Mp >= HWo + Wo + 1

    xr = x.reshape(N, Cin, D * HW_in)

    l_idx = jnp.arange(HW_in)[:, None]
    k_idx = jnp.arange(HWo)[None, :]
    sel = (l_idx == 2 * W * (k_idx // Wo) + 2 * (k_idx % Wo)).astype(jnp.bfloat16)
    j = jnp.arange(Wn)
    r = (j - HWo) % HWo
    w_i = r % Wo
    h_i = r // Wo
    rows = []
    for kh in range(3):
        for kw in range(3):
            ok = ((h_i + kh - 1 >= 0) & (h_i + kh - 1 < Ho)
                  & (w_i + kw - 1 >= 0) & (w_i + kw - 1 < Wo))
            rows.append(ok)
    mask = jnp.stack(rows, axis=0).astype(jnp.bfloat16)
    lane = jnp.arange(L)
    valid = ((lane >= HP) & (lane < HP + M)).astype(jnp.float32).reshape(1, L)

    w1f = _fold_w9(w1)
    w2f = _fold_w9(w2)

    kf = functools.partial(_fused_convs_kernel, Cin=Cin, C1=C1, Do=Do,
                           HW_in=HW_in, W_in=W, HWo=HWo, Wo=Wo, HP=HP,
                           M=M, Mp=Mp, NB=N, eps=1e-5)
    y2, s2, q2 = pl.pallas_call(
        kf,
        out_shape=(jax.ShapeDtypeStruct((N, C2, Mp), jnp.bfloat16),
                   jax.ShapeDtypeStruct((N, C2, 1), jnp.float32),
                   jax.ShapeDtypeStruct((N, C2, 1), jnp.float32)),
        grid=(2, N),
        in_specs=[pl.BlockSpec((1, Cin, D * HW_in),
                               lambda s, i: (i * (1 - s), 0, 0)),
                  pl.BlockSpec((HW_in, HWo), lambda s, i: (0, 0)),
                  pl.BlockSpec((3, C1, 9 * Cin), lambda s, i: (0, 0, 0)),
                  pl.BlockSpec((3, C2, 9 * C1), lambda s, i: (0, 0, 0)),
                  pl.BlockSpec((9, Wn), lambda s, i: (0, 0)),
                  pl.BlockSpec((1, L), lambda s, i: (0, 0)),
                  pl.BlockSpec((C1, 1), lambda s, i: (0, 0)),
                  pl.BlockSpec((C1, 1), lambda s, i: (0, 0))],
        out_specs=(pl.BlockSpec((1, C2, Mp), lambda s, i: (i * s, 0, 0)),
                   pl.BlockSpec((1, C2, 1), lambda s, i: (i * s, 0, 0)),
                   pl.BlockSpec((1, C2, 1), lambda s, i: (i * s, 0, 0))),
        scratch_shapes=[pltpu.VMEM((Do * Cin, HW_in), jnp.bfloat16),
                        pltpu.VMEM((Cin, L), jnp.bfloat16),
                        pltpu.VMEM((C1, L), jnp.bfloat16),
                        pltpu.VMEM((9 * Cin, Wn), jnp.bfloat16),
                        pltpu.VMEM((9 * C1, Wn), jnp.bfloat16),
                        pltpu.VMEM((N * C1, L), jnp.bfloat16),
                        pltpu.VMEM((C1, 2), jnp.float32),
                        pltpu.VMEM((C1, 2), jnp.float32)],
        compiler_params=pltpu.CompilerParams(
            dimension_semantics=("arbitrary", "arbitrary")),
    )(xr, sel, w1f, w2f, mask, valid, g1.reshape(-1, 1).astype(jnp.float32),
      be1.reshape(-1, 1).astype(jnp.float32))
    sc2, sh2 = _fold_bn(s2, q2, N * M, g2, be2)

    out = pl.pallas_call(
        _bn_relu_out_kernel,
        out_shape=jax.ShapeDtypeStruct((N, C2, Mp), jnp.float32),
        grid=(N,),
        in_specs=[pl.BlockSpec((1, C2, Mp), lambda n: (n, 0, 0)),
                  pl.BlockSpec((C2, 1), lambda n: (0, 0)),
                  pl.BlockSpec((C2, 1), lambda n: (0, 0))],
        out_specs=pl.BlockSpec((1, C2, Mp), lambda n: (n, 0, 0)),
        compiler_params=pltpu.CompilerParams(
            dimension_semantics=("arbitrary",)),
    )(y2, sc2, sh2)
    return out[:, :, :M].reshape(N, C2, Do, Ho, Wo)


# speedup vs baseline: 1.2968x; 1.0042x over previous
import functools

import jax
import jax.numpy as jnp
from jax.experimental import pallas as pl
from jax.experimental.pallas import tpu as pltpu


def _rup(x, m):
    return ((x + m - 1) // m) * m


def _cols9_dot(xs_ref, cols_ref, w_ref, mask_ref, *, C, HWo, Wo, HP, Mp, full):
    G = HWo
    Wn = Mp + 2 * G
    base = HP - G
    for kh in range(3):
        for kw in range(3):
            t = 3 * kh + kw
            sh = (kh - 1) * Wo + (kw - 1)
            tap = xs_ref[:, base + sh:base + sh + Wn]
            if not (full and t == 4):
                tap = tap * mask_ref[t:t + 1, :]
            cols_ref[t * C:(t + 1) * C, :] = tap
    acc = jnp.dot(w_ref[0], cols_ref[:, 0:Mp],
                  preferred_element_type=jnp.float32)
    acc += jnp.dot(w_ref[1], cols_ref[:, G:G + Mp],
                   preferred_element_type=jnp.float32)
    acc += jnp.dot(w_ref[2], cols_ref[:, 2 * G:2 * G + Mp],
                   preferred_element_type=jnp.float32)
    return acc


def _pool_conv1_kernel(x_ref, sel_ref, w_ref, mask_ref,
                       y_ref, ssum_ref, ssq_ref,
                       pool_ref, xs_ref, cols_ref, *,
                       C, Do, HW_in, W_in, HWo, Wo, HP, M, Mp, full):
    for do in range(Do):
        v = x_ref[0, :, 2 * HW_in * do:2 * HW_in * (do + 1)]
        a = jnp.maximum(v[:, :HW_in], v[:, HW_in:])
        b = jnp.maximum(a, jnp.concatenate([a[:, W_in:], a[:, :W_in]], axis=1))
        c = jnp.maximum(b, jnp.concatenate([b[:, 1:], b[:, :1]], axis=1))
        pool_ref[do * C:(do + 1) * C, :] = c.astype(jnp.bfloat16)
    p = jnp.dot(pool_ref[...], sel_ref[...],
                preferred_element_type=jnp.float32)
    p = p.astype(jnp.bfloat16)

    L = xs_ref.shape[1]
    xs_ref[:, :HP] = jnp.zeros((C, HP), jnp.bfloat16)
    for do in range(Do):
        xs_ref[:, HP + do * HWo:HP + (do + 1) * HWo] = p[do * C:(do + 1) * C, :]
    xs_ref[:, HP + M:] = jnp.zeros((C, L - HP - M), jnp.bfloat16)

    acc = _cols9_dot(xs_ref, cols_ref, w_ref, mask_ref,
                     C=C, HWo=HWo, Wo=Wo, HP=HP, Mp=Mp, full=full)
    cout = y_ref.shape[1]
    y_ref[0, :, :HP] = jnp.zeros((cout, HP), jnp.bfloat16)
    y_ref[0, :, HP:HP + Mp] = acc.astype(jnp.bfloat16)
    y_ref[0, :, HP + Mp:] = jnp.zeros((cout, L - HP - Mp), jnp.bfloat16)
    ssum_ref[0] = jnp.sum(acc, axis=1, keepdims=True)
    ssq_ref[0] = jnp.sum(acc * acc, axis=1, keepdims=True)


def _conv2_kernel(y1_ref, scale_ref, shift_ref, valid_ref, w_ref, mask_ref,
                  y2_ref, ssum_ref, ssq_ref,
                  xs_ref, cols_ref, *, C, HWo, Wo, HP, Mp, full):
    yv = y1_ref[0].astype(jnp.float32)
    act = jnp.maximum(yv * scale_ref[...] + shift_ref[...], 0.0)
    xs_ref[...] = (act * valid_ref[...]).astype(jnp.bfloat16)
    acc = _cols9_dot(xs_ref, cols_ref, w_ref, mask_ref,
                     C=C, HWo=HWo, Wo=Wo, HP=HP, Mp=Mp, full=full)
    y2_ref[0] = acc.astype(jnp.bfloat16)
    ssum_ref[0] = jnp.sum(acc, axis=1, keepdims=True)
    ssq_ref[0] = jnp.sum(acc * acc, axis=1, keepdims=True)


def _bn_relu_out_kernel(y_ref, scale_ref, shift_ref, o_ref):
    o_ref[0] = jnp.maximum(
        y_ref[0].astype(jnp.float32) * scale_ref[...] + shift_ref[...], 0.0)


def _fold_w9(w):
    cout, cin = w.shape[0], w.shape[1]
    wt = jnp.transpose(w.astype(jnp.float32), (2, 3, 4, 0, 1))
    wt = jnp.transpose(wt.reshape(3, 9, cout, cin), (0, 2, 1, 3))
    return wt.reshape(3, cout, 9 * cin).astype(jnp.bfloat16)


def _fold_bn(ssum, ssq, count, gamma, beta, eps=1e-5):
    s = jnp.sum(ssum[:, :, 0], axis=0)
    sq = jnp.sum(ssq[:, :, 0], axis=0)
    mean = s / count
    var = sq / count - mean * mean
    inv = gamma / jnp.sqrt(var + eps)
    scale = inv.reshape(-1, 1).astype(jnp.float32)
    shift = (beta - mean * inv).reshape(-1, 1).astype(jnp.float32)
    return scale, shift


def kernel(x, w1, g1, be1, w2, g2, be2):
    N, Cin, D, H, W = x.shape
    C1, C2 = w1.shape[0], w2.shape[0]
    assert Cin % 8 == 0 and C1 % 8 == 0 and C2 % 8 == 0 and N % 2 == 0
    assert D % 2 == 0 and H % 2 == 0 and W % 2 == 0
    Do, Ho, Wo = D // 2, H // 2, W // 2
    HWo = Ho * Wo
    M = Do * HWo
    Mp = _rup(M, 128)
    HP = _rup(HWo + Wo + 1, 128)
    L = HP + Mp + HP
    HW_in = H * W
    Wn = Mp + 2 * HWo
    full = (M == Mp)
    assert full
    assert HP >= HWo + Wo + 1 and L - HP - Mp >= HWo + Wo + 1

    xr = x.reshape(N, Cin, D * HW_in)

    l_idx = jnp.arange(HW_in)[:, None]
    k_idx = jnp.arange(HWo)[None, :]
    sel = (l_idx == 2 * W * (k_idx // Wo) + 2 * (k_idx % Wo)).astype(jnp.bfloat16)
    j = jnp.arange(Wn)
    r = (j - HWo) % HWo
    w_i = r % Wo
    h_i = r // Wo
    rows = []
    for kh in range(3):
        for kw in range(3):
            ok = ((h_i + kh - 1 >= 0) & (h_i + kh - 1 < Ho)
                  & (w_i + kw - 1 >= 0) & (w_i + kw - 1 < Wo))
            rows.append(ok)
    mask = jnp.stack(rows, axis=0).astype(jnp.bfloat16)
    lane = jnp.arange(L)
    valid = ((lane >= HP) & (lane < HP + M)).astype(jnp.float32).reshape(1, L)

    w1f = _fold_w9(w1)
    w2f = _fold_w9(w2)

    k1 = functools.partial(_pool_conv1_kernel, C=Cin, Do=Do, HW_in=HW_in,
                           W_in=W, HWo=HWo, Wo=Wo, HP=HP, M=M, Mp=Mp, full=full)
    y1, s1, q1 = pl.pallas_call(
        k1,
        out_shape=(jax.ShapeDtypeStruct((N, C1, L), jnp.bfloat16),
                   jax.ShapeDtypeStruct((N, C1, 1), jnp.float32),
                   jax.ShapeDtypeStruct((N, C1, 1), jnp.float32)),
        grid=(N,),
        in_specs=[pl.BlockSpec((1, Cin, D * HW_in), lambda n: (n, 0, 0)),
                  pl.BlockSpec((HW_in, HWo), lambda n: (0, 0)),
                  pl.BlockSpec((3, C1, 9 * Cin), lambda n: (0, 0, 0)),
                  pl.BlockSpec((9, Wn), lambda n: (0, 0))],
        out_specs=(pl.BlockSpec((1, C1, L), lambda n: (n, 0, 0)),
                   pl.BlockSpec((1, C1, 1), lambda n: (n, 0, 0)),
                   pl.BlockSpec((1, C1, 1), lambda n: (n, 0, 0))),
        scratch_shapes=[pltpu.VMEM((Do * Cin, HW_in), jnp.bfloat16),
                        pltpu.VMEM((Cin, L), jnp.bfloat16),
                        pltpu.VMEM((9 * Cin, Wn), jnp.bfloat16)],
        compiler_params=pltpu.CompilerParams(
            dimension_semantics=("arbitrary",)),
    )(xr, sel, w1f, mask)
    sc1, sh1 = _fold_bn(s1, q1, N * M, g1, be1)

    k2 = functools.partial(_conv2_kernel, C=C1, HWo=HWo, Wo=Wo, HP=HP, Mp=Mp,
                           full=full)
    y2, s2, q2 = pl.pallas_call(
        k2,
        out_shape=(jax.ShapeDtypeStruct((N, C2, Mp), jnp.bfloat16),
                   jax.ShapeDtypeStruct((N, C2, 1), jnp.float32),
                   jax.ShapeDtypeStruct((N, C2, 1), jnp.float32)),
        grid=(N,),
        in_specs=[pl.BlockSpec((1, C1, L), lambda n: (n, 0, 0)),
                  pl.BlockSpec((C1, 1), lambda n: (0, 0)),
                  pl.BlockSpec((C1, 1), lambda n: (0, 0)),
                  pl.BlockSpec((1, L), lambda n: (0, 0)),
                  pl.BlockSpec((3, C2, 9 * C1), lambda n: (0, 0, 0)),
                  pl.BlockSpec((9, Wn), lambda n: (0, 0))],
        out_specs=(pl.BlockSpec((1, C2, Mp), lambda n: (n, 0, 0)),
                   pl.BlockSpec((1, C2, 1), lambda n: (n, 0, 0)),
                   pl.BlockSpec((1, C2, 1), lambda n: (n, 0, 0))),
        scratch_shapes=[pltpu.VMEM((C1, L), jnp.bfloat16),
                        pltpu.VMEM((9 * C1, Wn), jnp.bfloat16)],
        compiler_params=pltpu.CompilerParams(
            dimension_semantics=("arbitrary",)),
    )(y1, sc1, sh1, valid, w2f, mask)
    sc2, sh2 = _fold_bn(s2, q2, N * M, g2, be2)

    out = pl.pallas_call(
        _bn_relu_out_kernel,
        out_shape=jax.ShapeDtypeStruct((N, C2, Mp), jnp.float32),
        grid=(N,),
        in_specs=[pl.BlockSpec((1, C2, Mp), lambda n: (n, 0, 0)),
                  pl.BlockSpec((C2, 1), lambda n: (0, 0)),
                  pl.BlockSpec((C2, 1), lambda n: (0, 0))],
        out_specs=pl.BlockSpec((1, C2, Mp), lambda n: (n, 0, 0)),
        compiler_params=pltpu.CompilerParams(
            dimension_semantics=("arbitrary",)),
    )(y2, sc2, sh2)
    return out[:, :, :M].reshape(N, C2, Do, Ho, Wo)
